# baseline (speedup 1.0000x reference)
import functools

import jax
import jax.numpy as jnp
from jax import lax
from jax.experimental import pallas as pl
from jax.experimental.pallas import tpu as pltpu
from jax.experimental.pallas import tpu_sc as plsc

VOCAB = 100000
MAXLEN = 2048
EMBED = 64
BATCH = 64
SEQ = 2048

NUM_CORES = 2
NUM_SUBCORES = 16
NW = NUM_CORES * NUM_SUBCORES
EPW = EMBED // NW
LANES = 16
NSTEP = SEQ // LANES


def _make_kernel():
    mesh = plsc.VectorSubcoreMesh(core_axis_name="c", subcore_axis_name="s")

    @functools.partial(
        pl.kernel,
        mesh=mesh,
        out_type=jax.ShapeDtypeStruct((BATCH, EMBED, SEQ), jnp.float32),
        compiler_params=pltpu.CompilerParams(
            use_tc_tiling_on_sc=True, needs_layout_passes=False),
        scratch_types=[
            pltpu.VMEM((VOCAB,), jnp.float32),
            pltpu.VMEM((SEQ,), jnp.int32),
            pltpu.VMEM((SEQ,), jnp.int32),
            pltpu.VMEM((SEQ,), jnp.float32),
            pltpu.VMEM((SEQ,), jnp.float32),
            pltpu.VMEM((SEQ,), jnp.float32),
        ]
        + [pltpu.SemaphoreType.DMA] * 6,
    )
    def emb(x_hbm, tokT_hbm, posT_hbm, outT_hbm, row_v, x0_v, x1_v, o0_v,
            o1_v, pos_r, *sems):
        xsem = sems[0:2]
        osem = sems[2:4]
        rsem = sems[4]
        psem = sems[5]
        xbufs = (x0_v, x1_v)
        obufs = (o0_v, o1_v)
        c = lax.axis_index("c")
        s = lax.axis_index("s")
        wid = s * NUM_CORES + c

        def per_component(t, carry):
            e = wid * EPW + t
            row_cp = pltpu.async_copy(tokT_hbm.at[e], row_v, rsem)
            pos_cp = pltpu.async_copy(posT_hbm.at[e], pos_r, psem)
            xfetch = {0: pltpu.async_copy(x_hbm.at[0], xbufs[0], xsem[0])}
            row_cp.wait()
            pos_cp.wait()

            stores = {}
            for b in range(BATCH):
                xb = b % 2
                xfetch[b].wait()
                if b + 1 < BATCH:
                    xfetch[b + 1] = pltpu.async_copy(
                        x_hbm.at[b + 1], xbufs[1 - xb], xsem[1 - xb])
                if b - 2 >= 0:
                    stores[b - 2].wait()
                x_v = xbufs[xb]
                o_v = obufs[xb]

                @plsc.parallel_loop(0, NSTEP, unroll=16)
                def sbody(i):
                    sl = pl.ds(i * LANES, LANES)
                    g = plsc.load_gather(row_v, [x_v[sl]])
                    o_v[sl] = g + pos_r[sl]
                stores[b] = pltpu.async_copy(o_v, outT_hbm.at[b, e, :],
                                             osem[xb])
            stores[BATCH - 2].wait()
            stores[BATCH - 1].wait()
            return carry

        lax.fori_loop(0, EPW, per_component, 0)

    return emb


_emb = _make_kernel()


def kernel(x, token_table, pos_table):
    outT = _emb(x.astype(jnp.int32), token_table.T, pos_table.T)
    return outT.transpose(0, 2, 1)


# speedup vs baseline: 1.3995x; 1.3995x over previous
import functools

import jax
import jax.numpy as jnp
from jax import lax
from jax.experimental import pallas as pl
from jax.experimental.pallas import tpu as pltpu
from jax.experimental.pallas import tpu_sc as plsc

VOCAB = 100000
MAXLEN = 2048
EMBED = 64
BATCH = 64
SEQ = 2048

NUM_CORES = 2
NUM_SUBCORES = 16
NW = NUM_CORES * NUM_SUBCORES
EPW = EMBED // NW
LANES = 16
NSTEP = SEQ // LANES
NBUF = 3


def _make_kernel():
    mesh = plsc.VectorSubcoreMesh(core_axis_name="c", subcore_axis_name="s")

    @functools.partial(
        pl.kernel,
        mesh=mesh,
        out_type=jax.ShapeDtypeStruct((BATCH, EMBED, SEQ), jnp.float32),
        compiler_params=pltpu.CompilerParams(
            use_tc_tiling_on_sc=True, needs_layout_passes=False),
        scratch_types=[
            pltpu.VMEM((VOCAB,), jnp.float32),
            pltpu.VMEM((SEQ,), jnp.int32),
            pltpu.VMEM((SEQ,), jnp.int32),
            pltpu.VMEM((SEQ,), jnp.int32),
            pltpu.VMEM((SEQ,), jnp.float32),
            pltpu.VMEM((SEQ,), jnp.float32),
            pltpu.VMEM((SEQ,), jnp.float32),
            pltpu.VMEM((SEQ,), jnp.float32),
        ]
        + [pltpu.SemaphoreType.DMA] * (2 * NBUF + 2),
    )
    def emb(x_hbm, tokT_hbm, posT_hbm, outT_hbm, row_v, xv0, xv1, xv2,
            ov0, ov1, ov2, pos_r, *sems):
        xbufs = (xv0, xv1, xv2)
        obufs = (ov0, ov1, ov2)
        xsem = sems[0:NBUF]
        osem = sems[NBUF : 2 * NBUF]
        rsem = sems[2 * NBUF]
        psem = sems[2 * NBUF + 1]
        c = lax.axis_index("c")
        s = lax.axis_index("s")
        wid = s * NUM_CORES + c

        def per_component(t, carry):
            e = wid * EPW + t
            row_cp = pltpu.async_copy(tokT_hbm.at[e], row_v, rsem)
            pos_cp = pltpu.async_copy(posT_hbm.at[e], pos_r, psem)

            def fetch_x(b):
                return pltpu.async_copy(
                    x_hbm.at[pl.ds(b * SEQ, SEQ)], xbufs[b % NBUF],
                    xsem[b % NBUF])

            xfetch = {0: fetch_x(0), 1: fetch_x(1)}
            row_cp.wait()
            pos_cp.wait()

            stores = {}
            for b in range(BATCH):
                xb = b % NBUF
                xfetch[b].wait()
                if b + 2 < BATCH:
                    xfetch[b + 2] = fetch_x(b + 2)
                if b - NBUF >= 0:
                    stores[b - NBUF].wait()
                xr = xbufs[xb]
                orow = obufs[xb]

                @plsc.parallel_loop(0, NSTEP, unroll=8)
                def sbody(i):
                    sl = pl.ds(i * LANES, LANES)
                    g = plsc.load_gather(row_v, [xr[sl]])
                    orow[sl] = g + pos_r[sl]

                stores[b] = pltpu.async_copy(orow, outT_hbm.at[b, e, :],
                                             osem[xb])
            for b in range(BATCH - NBUF, BATCH):
                stores[b].wait()
            return carry

        lax.fori_loop(0, EPW, per_component, 0)

    return emb


_emb = _make_kernel()


def kernel(x, token_table, pos_table):
    x_flat = x.reshape(BATCH * SEQ).astype(jnp.int32)
    outT = _emb(x_flat, token_table.T, pos_table.T)
    return outT.transpose(0, 2, 1)
